# baseline (speedup 1.0000x reference)
import functools

import jax
import jax.numpy as jnp
from jax import lax
from jax.experimental import pallas as pl
from jax.experimental.pallas import tpu as pltpu
from jax.experimental.pallas import tpu_sc as plsc

B = 16384
D = 64
XD = 16
NC = 2
NS = 16
NW = NC * NS
BPW = B // NW
CS = 16
NCH = BPW // CS

_mesh = plsc.VectorSubcoreMesh(core_axis_name="c", subcore_axis_name="s")


@functools.partial(
    pl.kernel,
    mesh=_mesh,
    out_type=(
        jax.ShapeDtypeStruct((B, XD), jnp.float32),
        jax.ShapeDtypeStruct((B, XD), jnp.float32),
    ),
    scratch_types=[
        pltpu.VMEM((BPW,), jnp.int32),
        pltpu.VMEM((BPW,), jnp.int32),
        pltpu.VMEM((BPW,), jnp.float32),
        pltpu.VMEM((1, XD), jnp.float32),
        pltpu.VMEM((CS, 2 * D), jnp.float32),
        pltpu.VMEM((CS, 2 * D), jnp.float32),
        pltpu.VMEM((CS, 2 * D), jnp.float32),
        pltpu.VMEM((CS, 2 * D), jnp.float32),
        pltpu.VMEM((CS, XD), jnp.float32),
        pltpu.VMEM((CS, XD), jnp.float32),
        pltpu.VMEM((CS, XD), jnp.float32),
        pltpu.VMEM((CS, XD), jnp.float32),
        pltpu.SemaphoreType.DMA,
        pltpu.SemaphoreType.DMA,
        pltpu.SemaphoreType.DMA,
        pltpu.SemaphoreType.DMA,
        pltpu.SemaphoreType.DMA,
        pltpu.SemaphoreType.DMA,
    ],
)
def _sc_partials(users_hbm, items_hbm, xij_hbm, utab_hbm, itab_hbm, wtab_hbm,
                 nume_hbm, deno_hbm, uidx_v, iidx_v, x_v, w_v,
                 ub0, ub1, ib0, ib1, on0, on1, od0, od1,
                 semu0, semu1, semi0, semi1, semo0, semo1):
    wid = lax.axis_index("s") * NC + lax.axis_index("c")
    base = wid * BPW
    pltpu.sync_copy(users_hbm.at[pl.ds(base, BPW)], uidx_v)
    pltpu.sync_copy(items_hbm.at[pl.ds(base, BPW)], iidx_v)
    pltpu.sync_copy(xij_hbm.at[pl.ds(base, BPW)], x_v)
    pltpu.sync_copy(wtab_hbm, w_v)

    ubufs = (ub0, ub1)
    ibufs = (ib0, ib1)
    usems = (semu0, semu1)
    isems = (semi0, semi1)
    onbufs = (on0, on1)
    odbufs = (od0, od1)
    osems = (semo0, semo1)

    def issue(c, b):
        sl = pl.ds(c * CS, CS)
        upr = lax.shift_right_logical(uidx_v[sl], 1)
        ipr = lax.shift_right_logical(iidx_v[sl], 1)
        for ri in range(CS):
            pltpu.async_copy(utab_hbm.at[upr[ri]], ubufs[b].at[ri], usems[b])
            pltpu.async_copy(itab_hbm.at[ipr[ri]], ibufs[b].at[ri], isems[b])

    def wait(b):
        pltpu.make_async_copy(utab_hbm.at[pl.ds(0, CS)], ubufs[b],
                              usems[b]).wait()
        pltpu.make_async_copy(itab_hbm.at[pl.ds(0, CS)], ibufs[b],
                              isems[b]).wait()

    issue(0, 0)
    issue(1, 1)

    w = w_v[0]

    def wait_out(c, b):
        pltpu.make_async_copy(onbufs[b], nume_hbm.at[pl.ds(0, CS)],
                              osems[b]).wait()
        pltpu.make_async_copy(odbufs[b], deno_hbm.at[pl.ds(0, CS)],
                              osems[b]).wait()

    def chunk_compute(c, b):
        @pl.when(c >= 2)
        def _():
            wait_out(c, b)

        sl = pl.ds(c * CS, CS)
        t_vec = x_v[sl] - 0.3
        pu_vec = jnp.bitwise_and(uidx_v[sl], 1) * D
        pi_vec = jnp.bitwise_and(iidx_v[sl], 1) * D
        for ri in range(CS):
            r = c * CS + ri
            pu = pu_vec[ri]
            pi = pi_vec[ri]
            xe = w * t_vec[ri]
            e = jnp.exp(xe)
            nume = (e / (1.0 + e)) * e
            deno = e
            for j in range(D // 16):
                u = ubufs[b][ri, pl.ds(pu + j * 16, 16)]
                v = ibufs[b][ri, pl.ds(pi + j * 16, 16)]
                ev = jnp.exp(v)
                s = 1.0 / (1.0 + jnp.exp(-u))
                nume = nume + s * ev
                deno = deno + ev
            onbufs[b][ri, pl.ds(0, XD)] = nume
            odbufs[b][ri, pl.ds(0, XD)] = deno
        osl = pl.ds(base + c * CS, CS)
        pltpu.async_copy(onbufs[b], nume_hbm.at[osl], osems[b])
        pltpu.async_copy(odbufs[b], deno_hbm.at[osl], osems[b])

    def pipe_body(g, carry):
        for b in (0, 1):
            c = 2 * g + b
            wait(b)
            chunk_compute(c, b)

            @pl.when(c + 2 < NCH)
            def _():
                issue(c + 2, b)

        return carry

    lax.fori_loop(0, NCH // 2, pipe_body, 0)
    wait_out(NCH, 0)
    wait_out(NCH, 1)


_RB = 512


def _tc_reduce_body(nume_ref, deno_ref, out_ref):
    r_iota = lax.broadcasted_iota(jnp.int32, (128, 8), 0)
    c_iota = lax.broadcasted_iota(jnp.int32, (128, 8), 1)
    sel = jnp.where(r_iota // XD == c_iota, 1.0, 0.0).astype(jnp.float32)
    ns = jnp.dot(nume_ref[...], sel, preferred_element_type=jnp.float32)
    ds = jnp.dot(deno_ref[...], sel, preferred_element_type=jnp.float32)
    out_ref[...] = ns / ds


_tc_reduce = pl.pallas_call(
    _tc_reduce_body,
    grid=(B // 8 // _RB,),
    in_specs=[
        pl.BlockSpec((_RB, 128), lambda i: (i, 0)),
        pl.BlockSpec((_RB, 128), lambda i: (i, 0)),
    ],
    out_specs=pl.BlockSpec((_RB, 8), lambda i: (i, 0)),
    out_shape=jax.ShapeDtypeStruct((B // 8, 8), jnp.float32),
)


def kernel(users, items, xij, user_table, item_table, xij_table):
    utab2 = user_table.reshape(1000000 // 2, 2 * D)
    itab2 = item_table.reshape(1000000 // 2, 2 * D)
    nume, deno = _sc_partials(users, items, xij, utab2, itab2, xij_table)
    rating = _tc_reduce(nume.reshape(B // 8, 128), deno.reshape(B // 8, 128))
    return rating.reshape(B)


# speedup vs baseline: 2.1828x; 2.1828x over previous
import functools

import jax
import jax.numpy as jnp
from jax import lax
from jax.experimental import pallas as pl
from jax.experimental.pallas import tpu as pltpu
from jax.experimental.pallas import tpu_sc as plsc

B = 16384
D = 64
XD = 16
NC = 2
NS = 16
NW = NC * NS
BPW = B // NW
CS = 16
NCH = BPW // CS

_mesh = plsc.VectorSubcoreMesh(core_axis_name="c", subcore_axis_name="s")


@functools.partial(
    pl.kernel,
    mesh=_mesh,
    out_type=(
        jax.ShapeDtypeStruct((B, XD), jnp.float32),
        jax.ShapeDtypeStruct((B, XD), jnp.float32),
    ),
    scratch_types=[
        pltpu.VMEM((BPW,), jnp.int32),
        pltpu.VMEM((BPW,), jnp.int32),
        pltpu.VMEM((BPW,), jnp.float32),
        pltpu.VMEM((1, XD), jnp.float32),
        pltpu.VMEM((CS, 8, D), jnp.float32),
        pltpu.VMEM((CS, 8, D), jnp.float32),
        pltpu.VMEM((CS, 8, D), jnp.float32),
        pltpu.VMEM((CS, 8, D), jnp.float32),
        pltpu.VMEM((CS, XD), jnp.float32),
        pltpu.VMEM((CS, XD), jnp.float32),
        pltpu.VMEM((CS, XD), jnp.float32),
        pltpu.VMEM((CS, XD), jnp.float32),
        pltpu.SemaphoreType.DMA,
        pltpu.SemaphoreType.DMA,
        pltpu.SemaphoreType.DMA,
        pltpu.SemaphoreType.DMA,
        pltpu.SemaphoreType.DMA,
        pltpu.SemaphoreType.DMA,
    ],
)
def _sc_partials(users_hbm, items_hbm, xij_hbm, utab_hbm, itab_hbm, wtab_hbm,
                 nume_hbm, deno_hbm, uidx_v, iidx_v, x_v, w_v,
                 ub0, ub1, ib0, ib1, on0, on1, od0, od1,
                 semu0, semu1, semi0, semi1, semo0, semo1):
    wid = lax.axis_index("s") * NC + lax.axis_index("c")
    base = wid * BPW
    pltpu.sync_copy(users_hbm.at[pl.ds(base, BPW)], uidx_v)
    pltpu.sync_copy(items_hbm.at[pl.ds(base, BPW)], iidx_v)
    pltpu.sync_copy(xij_hbm.at[pl.ds(base, BPW)], x_v)
    pltpu.sync_copy(wtab_hbm, w_v)

    ubufs = (ub0, ub1)
    ibufs = (ib0, ib1)
    usems = (semu0, semu1)
    isems = (semi0, semi1)
    onbufs = (on0, on1)
    odbufs = (od0, od1)
    osems = (semo0, semo1)

    def issue(c, b):
        sl = pl.ds(c * CS, CS)
        utvec = lax.shift_right_logical(uidx_v[sl], 3)
        itvec = lax.shift_right_logical(iidx_v[sl], 3)
        for ri in range(CS):
            pltpu.async_copy(utab_hbm.at[utvec[ri]], ubufs[b].at[ri],
                             usems[b])
            pltpu.async_copy(itab_hbm.at[itvec[ri]], ibufs[b].at[ri],
                             isems[b])

    def wait(b):
        pltpu.make_async_copy(utab_hbm.at[pl.ds(0, CS)], ubufs[b],
                              usems[b]).wait()
        pltpu.make_async_copy(itab_hbm.at[pl.ds(0, CS)], ibufs[b],
                              isems[b]).wait()

    issue(0, 0)
    issue(1, 1)

    w = w_v[0]

    def wait_out(c, b):
        pltpu.make_async_copy(onbufs[b], nume_hbm.at[pl.ds(0, CS)],
                              osems[b]).wait()
        pltpu.make_async_copy(odbufs[b], deno_hbm.at[pl.ds(0, CS)],
                              osems[b]).wait()

    def chunk_compute(c, b):
        @pl.when(c >= 2)
        def _():
            wait_out(c, b)

        sl = pl.ds(c * CS, CS)
        t_vec = x_v[sl] - 0.3
        pu_vec = jnp.bitwise_and(uidx_v[sl], 7)
        pi_vec = jnp.bitwise_and(iidx_v[sl], 7)
        for ri in range(CS):
            r = c * CS + ri
            pu = pu_vec[ri]
            pi = pi_vec[ri]
            xe = w * t_vec[ri]
            e = jnp.exp(xe)
            nume = (e / (1.0 + e)) * e
            deno = e
            for j in range(D // 16):
                u = ubufs[b][ri, pu, pl.ds(j * 16, 16)]
                v = ibufs[b][ri, pi, pl.ds(j * 16, 16)]
                ev = jnp.exp(v)
                s = 1.0 / (1.0 + jnp.exp(-u))
                nume = nume + s * ev
                deno = deno + ev
            onbufs[b][ri, pl.ds(0, XD)] = nume
            odbufs[b][ri, pl.ds(0, XD)] = deno
        osl = pl.ds(base + c * CS, CS)
        pltpu.async_copy(onbufs[b], nume_hbm.at[osl], osems[b])
        pltpu.async_copy(odbufs[b], deno_hbm.at[osl], osems[b])

    def pipe_body(g, carry):
        for b in (0, 1):
            c = 2 * g + b
            wait(b)
            chunk_compute(c, b)

            @pl.when(c + 2 < NCH)
            def _():
                issue(c + 2, b)

        return carry

    lax.fori_loop(0, NCH // 2, pipe_body, 0)
    wait_out(NCH, 0)
    wait_out(NCH, 1)


_RB = 512


def _tc_reduce_body(nume_ref, deno_ref, out_ref):
    r_iota = lax.broadcasted_iota(jnp.int32, (128, 8), 0)
    c_iota = lax.broadcasted_iota(jnp.int32, (128, 8), 1)
    sel = jnp.where(r_iota // XD == c_iota, 1.0, 0.0).astype(jnp.float32)
    ns = jnp.dot(nume_ref[...], sel, preferred_element_type=jnp.float32)
    ds = jnp.dot(deno_ref[...], sel, preferred_element_type=jnp.float32)
    out_ref[...] = ns / ds


_tc_reduce = pl.pallas_call(
    _tc_reduce_body,
    grid=(B // 8 // _RB,),
    in_specs=[
        pl.BlockSpec((_RB, 128), lambda i: (i, 0)),
        pl.BlockSpec((_RB, 128), lambda i: (i, 0)),
    ],
    out_specs=pl.BlockSpec((_RB, 8), lambda i: (i, 0)),
    out_shape=jax.ShapeDtypeStruct((B // 8, 8), jnp.float32),
)


def kernel(users, items, xij, user_table, item_table, xij_table):
    utab3 = user_table.reshape(1000000 // 8, 8, D)
    itab3 = item_table.reshape(1000000 // 8, 8, D)
    nume, deno = _sc_partials(users, items, xij, utab3, itab3, xij_table)
    rating = _tc_reduce(nume.reshape(B // 8, 128), deno.reshape(B // 8, 128))
    return rating.reshape(B)
